# baseline (speedup 1.0000x reference)
import functools

import jax
import jax.numpy as jnp
from jax import lax
from jax.experimental import pallas as pl
from jax.experimental.pallas import tpu as pltpu
from jax.experimental.pallas import tpu_sc as plsc

IN_F = 4096
OUT_F = 4096
TOTAL = IN_F * OUT_F

NW = 32
CSZ = 65536
CROWS = CSZ // IN_F
SHIFT = 12
SAMP = 512
NCH = TOTAL // CSZ
NSUB = NCH // NW
WIN = 8192
WBUF = WIN + 16
SBUF = ((NCH + 1 + 31) // 16) * 16


def _sel(buf, i):
    return buf[pl.ds(i, 16)][0]


def _scatter_body(rows_hbm, cols_hbm, vals_hbm, coarse_hbm, w_hbm,
                  sbuf, dense, rwin, cwin, vwin, sem_ld, sem_o):
    w = lax.axis_index("s") * 2 + lax.axis_index("c")

    pltpu.sync_copy(coarse_hbm, sbuf)

    @pl.loop(0, CROWS)
    def _zero_row(r):
        @pl.loop(0, IN_F // 16)
        def _zero_init(i):
            dense[r, pl.ds(i * 16, 16)] = jnp.zeros((16,), jnp.float32)

    def _windows(c, start, end, value_of):
        astart = jnp.bitwise_and(start, jnp.int32(-8))
        delta = start - astart
        rbase = c * CROWS
        nwin = (end - start + WIN - 1) // WIN

        def _win(m, carry):
            off = pl.multiple_of(astart + m * WIN, 8)
            ld1 = pltpu.async_copy(rows_hbm.at[pl.ds(off, WBUF)], rwin,
                                   sem_ld)
            ld2 = pltpu.async_copy(cols_hbm.at[pl.ds(off, WBUF)], cwin,
                                   sem_ld)
            ld3 = pltpu.async_copy(vals_hbm.at[pl.ds(off, WBUF)], vwin,
                                   sem_ld)
            ld1.wait()
            ld2.wait()
            ld3.wait()
            rem = end - start - m * WIN
            n_j = (jnp.minimum(rem, WIN) + 15) // 16
            wbase = start + m * WIN

            @pl.loop(0, n_j)
            def _scat(j):
                t = j * 16
                rv = rwin[pl.ds(delta + t, 16)]
                cv = cwin[pl.ds(delta + t, 16)]
                vv = vwin[pl.ds(delta + t, 16)]
                g = wbase + t + lax.broadcasted_iota(jnp.int32, (16,), 0)
                lrow = rv - rbase
                mask = ((g < end) & (lrow >= 0) & (lrow < CROWS))
                plsc.store_scatter(dense, [lrow, cv], value_of(vv),
                                   mask=mask)

            return carry

        lax.fori_loop(0, nwin, _win, 0)

    @pl.loop(0, NSUB)
    def _sub(s):
        c = w * NSUB + s
        co0 = _sel(sbuf, c)
        co1 = _sel(sbuf, c + 1)
        lo = jnp.maximum(co0 - 1, 0) * SAMP
        hi = co1 * SAMP
        with jax.named_scope("sc_scatter"):
            _windows(c, lo, hi, lambda v: v)
        with jax.named_scope("sc_dma_out"):
            r0 = pl.multiple_of(c * CROWS, CROWS)
            pltpu.async_copy(dense, w_hbm.at[pl.ds(r0, CROWS)],
                             sem_o).wait()
        with jax.named_scope("sc_clean"):
            _windows(c, lo, hi,
                     lambda v: jnp.zeros((16,), jnp.float32))


def _densify(rows_p, cols_p, vals_p, coarse):
    mesh = plsc.VectorSubcoreMesh(core_axis_name="c", subcore_axis_name="s")
    return pl.kernel(
        _scatter_body,
        out_type=jax.ShapeDtypeStruct((OUT_F, IN_F), jnp.float32),
        mesh=mesh,
        compiler_params=pltpu.CompilerParams(needs_layout_passes=False),
        scratch_types=[
            pltpu.VMEM((SBUF,), jnp.int32),
            pltpu.VMEM((CROWS, IN_F), jnp.float32),
            pltpu.VMEM((WBUF,), jnp.int32),
            pltpu.VMEM((WBUF,), jnp.int32),
            pltpu.VMEM((WBUF,), jnp.float32),
            pltpu.SemaphoreType.DMA,
            pltpu.SemaphoreType.DMA,
        ],
    )(rows_p, cols_p, vals_p, coarse)


def _mm_body(x_ref, w_ref, b_ref, o_ref):
    acc = lax.dot_general(
        x_ref[...], w_ref[...],
        (((1,), (1,)), ((), ())),
        preferred_element_type=jnp.float32)
    o_ref[...] = acc + b_ref[...][None, :]


def _matmul(x, w, bias, batch):
    nb = 512
    return pl.pallas_call(
        _mm_body,
        grid=(OUT_F // nb,),
        in_specs=[
            pl.BlockSpec((batch, IN_F), lambda j: (0, 0)),
            pl.BlockSpec((nb, IN_F), lambda j: (j, 0)),
            pl.BlockSpec((nb,), lambda j: (j,)),
        ],
        out_specs=pl.BlockSpec((batch, nb), lambda j: (0, j)),
        out_shape=jax.ShapeDtypeStruct((batch, OUT_F), jnp.float32),
    )(x, w, bias)


def kernel(x, sparse_indices, sparse_values, bias):
    orig_shape = x.shape
    x2d = x.reshape(-1, IN_F)
    batch = x2d.shape[0]

    nnz = sparse_values.shape[0]
    rows = sparse_indices[0]
    cols = sparse_indices[1]
    padn = -(-(nnz + 3 * WIN) // SAMP) * SAMP
    pad = padn - nnz
    rows_p = jnp.concatenate([rows, jnp.full((pad,), OUT_F, rows.dtype)])
    cols_p = jnp.concatenate([cols, jnp.zeros((pad,), cols.dtype)])
    vals_p = jnp.concatenate(
        [sparse_values, jnp.zeros((pad,), sparse_values.dtype)])
    ns = -(-nnz // SAMP)
    sample = jnp.min(rows_p[:ns * SAMP].reshape(ns, SAMP), axis=1)
    rbounds = jnp.arange(NCH + 1, dtype=rows.dtype) * CROWS
    coarse = jnp.sum(sample[None, :] < rbounds[:, None],
                     axis=1).astype(jnp.int32)
    coarse_p = jnp.concatenate(
        [coarse, jnp.zeros((SBUF - NCH - 1,), jnp.int32)])

    w = _densify(rows_p, cols_p, vals_p, coarse_p)
    y = _matmul(x2d, w, bias, batch)
    return y.reshape(*orig_shape[:-1], OUT_F).astype(x.dtype)


# speedup vs baseline: 1.3116x; 1.3116x over previous
import functools

import jax
import jax.numpy as jnp
from jax import lax
from jax.experimental import pallas as pl
from jax.experimental.pallas import tpu as pltpu
from jax.experimental.pallas import tpu_sc as plsc

IN_F = 4096
OUT_F = 4096
TOTAL = IN_F * OUT_F

NW = 32
CSZ = 65536
CROWS = CSZ // IN_F
SHIFT = 12
SAMP = 512
NCH = TOTAL // CSZ
NSUB = NCH // NW
WIN = 8192
WBUF = WIN + 16
SBUF = ((NCH + 1 + 31) // 16) * 16


def _sel(buf, i):
    return buf[pl.ds(i, 16)][0]


def _split_body(idx_ref, v_ref, r_ref, c_ref, vo_ref):
    r_ref[...] = idx_ref[0, :]
    c_ref[...] = idx_ref[1, :]
    vo_ref[...] = v_ref[...]


def _split(sparse_indices, sparse_values, padn):
    bn = 131072
    grid = (padn // bn,)
    return pl.pallas_call(
        _split_body,
        grid=grid,
        in_specs=[
            pl.BlockSpec((2, bn), lambda j: (0, j)),
            pl.BlockSpec((bn,), lambda j: (j,)),
        ],
        out_specs=[
            pl.BlockSpec((bn,), lambda j: (j,)),
            pl.BlockSpec((bn,), lambda j: (j,)),
            pl.BlockSpec((bn,), lambda j: (j,)),
        ],
        out_shape=[
            jax.ShapeDtypeStruct((padn,), sparse_indices.dtype),
            jax.ShapeDtypeStruct((padn,), sparse_indices.dtype),
            jax.ShapeDtypeStruct((padn,), sparse_values.dtype),
        ],
    )(sparse_indices, sparse_values)


def _scatter_body(nnz, rows_hbm, cols_hbm, vals_hbm, coarse_hbm, w_hbm,
                  sbuf, dense, rwin, cwin, vwin, sem_ld, sem_o):
    w = lax.axis_index("s") * 2 + lax.axis_index("c")

    pltpu.sync_copy(coarse_hbm, sbuf)

    @pl.loop(0, CROWS)
    def _zero_row(r):
        @pl.loop(0, IN_F // 16)
        def _zero_init(i):
            dense[r, pl.ds(i * 16, 16)] = jnp.zeros((16,), jnp.float32)

    def _windows(c, start, end, value_of):
        astart = jnp.bitwise_and(start, jnp.int32(-8))
        delta = start - astart
        rbase = c * CROWS
        nwin = (end - start + WIN - 1) // WIN

        def _win(m, carry):
            off = pl.multiple_of(astart + m * WIN, 8)
            ld1 = pltpu.async_copy(rows_hbm.at[pl.ds(off, WBUF)], rwin,
                                   sem_ld)
            ld2 = pltpu.async_copy(cols_hbm.at[pl.ds(off, WBUF)], cwin,
                                   sem_ld)
            ld3 = pltpu.async_copy(vals_hbm.at[pl.ds(off, WBUF)], vwin,
                                   sem_ld)
            ld1.wait()
            ld2.wait()
            ld3.wait()
            rem = end - start - m * WIN
            n_j = (jnp.minimum(rem, WIN) + 15) // 16
            wbase = start + m * WIN

            @pl.loop(0, n_j)
            def _scat(j):
                t = j * 16
                rv = rwin[pl.ds(delta + t, 16)]
                cv = cwin[pl.ds(delta + t, 16)]
                vv = vwin[pl.ds(delta + t, 16)]
                g = wbase + t + lax.broadcasted_iota(jnp.int32, (16,), 0)
                lrow = rv - rbase
                mask = ((g < end) & (lrow >= 0) & (lrow < CROWS))
                plsc.store_scatter(dense, [lrow, cv], value_of(vv),
                                   mask=mask)

            return carry

        lax.fori_loop(0, nwin, _win, 0)

    @pl.loop(0, NSUB)
    def _sub(s):
        c = w * NSUB + s
        co0 = _sel(sbuf, c)
        co1 = _sel(sbuf, c + 1)
        lo = jnp.maximum(co0 - 1, 0) * SAMP
        hi = jnp.minimum(co1 * SAMP, nnz)
        with jax.named_scope("sc_scatter"):
            _windows(c, lo, hi, lambda v: v)
        with jax.named_scope("sc_dma_out"):
            r0 = pl.multiple_of(c * CROWS, CROWS)
            pltpu.async_copy(dense, w_hbm.at[pl.ds(r0, CROWS)],
                             sem_o).wait()
        with jax.named_scope("sc_clean"):
            _windows(c, lo, hi,
                     lambda v: jnp.zeros((16,), jnp.float32))


def _densify(rows_p, cols_p, vals_p, coarse, nnz):
    mesh = plsc.VectorSubcoreMesh(core_axis_name="c", subcore_axis_name="s")
    return pl.kernel(
        functools.partial(_scatter_body, nnz),
        out_type=jax.ShapeDtypeStruct((OUT_F, IN_F), jnp.float32),
        mesh=mesh,
        compiler_params=pltpu.CompilerParams(needs_layout_passes=False),
        scratch_types=[
            pltpu.VMEM((SBUF,), jnp.int32),
            pltpu.VMEM((CROWS, IN_F), jnp.float32),
            pltpu.VMEM((WBUF,), jnp.int32),
            pltpu.VMEM((WBUF,), jnp.int32),
            pltpu.VMEM((WBUF,), jnp.float32),
            pltpu.SemaphoreType.DMA,
            pltpu.SemaphoreType.DMA,
        ],
    )(rows_p, cols_p, vals_p, coarse)


def _mm_body(x_ref, w_ref, b_ref, o_ref):
    acc = lax.dot_general(
        x_ref[...], w_ref[...],
        (((1,), (1,)), ((), ())),
        preferred_element_type=jnp.float32)
    o_ref[...] = acc + b_ref[...][None, :]


def _matmul(x, w, bias, batch):
    nb = 512
    return pl.pallas_call(
        _mm_body,
        grid=(OUT_F // nb,),
        in_specs=[
            pl.BlockSpec((batch, IN_F), lambda j: (0, 0)),
            pl.BlockSpec((nb, IN_F), lambda j: (j, 0)),
            pl.BlockSpec((nb,), lambda j: (j,)),
        ],
        out_specs=pl.BlockSpec((batch, nb), lambda j: (0, j)),
        out_shape=jax.ShapeDtypeStruct((batch, OUT_F), jnp.float32),
    )(x, w, bias)


def kernel(x, sparse_indices, sparse_values, bias):
    orig_shape = x.shape
    x2d = x.reshape(-1, IN_F)
    batch = x2d.shape[0]

    nnz = sparse_values.shape[0]
    padn = -(-(nnz + 3 * WIN) // 131072) * 131072
    rows_p, cols_p, vals_p = _split(sparse_indices, sparse_values, padn)
    ns = -(-nnz // SAMP)
    pos2d = (jnp.arange(ns, dtype=jnp.int32)[:, None] * SAMP
             + jnp.arange(SAMP, dtype=jnp.int32)[None, :])
    r2d = rows_p[:ns * SAMP].reshape(ns, SAMP)
    sample = jnp.min(jnp.where(pos2d < nnz, r2d, OUT_F), axis=1)
    rbounds = jnp.arange(NCH + 1, dtype=jnp.int32) * CROWS
    coarse = jnp.sum(sample[None, :] < rbounds[:, None],
                     axis=1).astype(jnp.int32)
    coarse_p = jnp.concatenate(
        [coarse, jnp.zeros((SBUF - NCH - 1,), jnp.int32)])

    w = _densify(rows_p, cols_p, vals_p, coarse_p, nnz)
    y = _matmul(x2d, w, bias, batch)
    return y.reshape(*orig_shape[:-1], OUT_F).astype(x.dtype)


# speedup vs baseline: 1.3846x; 1.0557x over previous
import functools

import jax
import jax.numpy as jnp
from jax import lax
from jax.experimental import pallas as pl
from jax.experimental.pallas import tpu as pltpu
from jax.experimental.pallas import tpu_sc as plsc

IN_F = 4096
OUT_F = 4096
TOTAL = IN_F * OUT_F

NW = 32
CSZ = 65536
CROWS = CSZ // IN_F
SHIFT = 12
SAMP = 512
NCH = TOTAL // CSZ
NSUB = NCH // NW
WIN = 8192
WBUF = WIN + 16
SBUF = ((NCH + 1 + 31) // 16) * 16


def _sel(buf, i):
    return buf[pl.ds(i, 16)][0]


def _split_body(idx_ref, v_ref, r_ref, c_ref, vo_ref):
    r_ref[...] = idx_ref[0, :]
    c_ref[...] = idx_ref[1, :]
    vo_ref[...] = v_ref[...]


def _split(sparse_indices, sparse_values, padn):
    bn = 131072
    grid = (padn // bn,)
    return pl.pallas_call(
        _split_body,
        grid=grid,
        in_specs=[
            pl.BlockSpec((2, bn), lambda j: (0, j)),
            pl.BlockSpec((bn,), lambda j: (j,)),
        ],
        out_specs=[
            pl.BlockSpec((bn,), lambda j: (j,)),
            pl.BlockSpec((bn,), lambda j: (j,)),
            pl.BlockSpec((bn,), lambda j: (j,)),
        ],
        out_shape=[
            jax.ShapeDtypeStruct((padn,), sparse_indices.dtype),
            jax.ShapeDtypeStruct((padn,), sparse_indices.dtype),
            jax.ShapeDtypeStruct((padn,), sparse_values.dtype),
        ],
    )(sparse_indices, sparse_values)


def _scatter_body(nnz, rows_hbm, cols_hbm, vals_hbm, coarse_hbm, w_hbm,
                  sbuf, dense, rwin, cwin, vwin, sem_ld, sem_o):
    w = lax.axis_index("s") * 2 + lax.axis_index("c")

    pltpu.sync_copy(coarse_hbm, sbuf)

    @pl.loop(0, CROWS)
    def _zero_row(r):
        @pl.loop(0, IN_F // 16)
        def _zero_init(i):
            dense[r, pl.ds(i * 16, 16)] = jnp.zeros((16,), jnp.float32)

    def _windows(c, start, end, value_of):
        astart = jnp.bitwise_and(start, jnp.int32(-8))
        delta = start - astart
        rbase = c * CROWS
        nwin = (end - start + WIN - 1) // WIN

        def _win(m, carry):
            off = pl.multiple_of(astart + m * WIN, 8)
            ld1 = pltpu.async_copy(rows_hbm.at[pl.ds(off, WBUF)], rwin,
                                   sem_ld)
            ld2 = pltpu.async_copy(cols_hbm.at[pl.ds(off, WBUF)], cwin,
                                   sem_ld)
            ld3 = pltpu.async_copy(vals_hbm.at[pl.ds(off, WBUF)], vwin,
                                   sem_ld)
            ld1.wait()
            ld2.wait()
            ld3.wait()
            rem = end - start - m * WIN
            n_j = (jnp.minimum(rem, WIN) + 15) // 16
            wbase = start + m * WIN

            @pl.loop(0, n_j)
            def _scat(j):
                t = j * 16
                rv = rwin[pl.ds(delta + t, 16)]
                cv = cwin[pl.ds(delta + t, 16)]
                vv = vwin[pl.ds(delta + t, 16)]
                g = wbase + t + lax.broadcasted_iota(jnp.int32, (16,), 0)
                lrow = rv - rbase
                mask = ((g < end) & (lrow >= 0) & (lrow < CROWS))
                plsc.store_scatter(dense, [lrow, cv], value_of(vv),
                                   mask=mask)

            return carry

        lax.fori_loop(0, nwin, _win, 0)

    @pl.loop(0, NSUB)
    def _sub(s):
        c = w * NSUB + s
        co0 = _sel(sbuf, c)
        co1 = _sel(sbuf, c + 1)
        lo = jnp.maximum(co0 - 1, 0) * SAMP
        hi = jnp.minimum(co1 * SAMP, nnz)
        r0 = pl.multiple_of(c * CROWS, CROWS)
        nwin = (hi - lo + WIN - 1) // WIN

        @pl.when(nwin <= 1)
        def _fast():
            astart = jnp.bitwise_and(lo, jnp.int32(-8))
            delta = lo - astart
            rbase = c * CROWS
            off = pl.multiple_of(astart, 8)
            ld1 = pltpu.async_copy(rows_hbm.at[pl.ds(off, WBUF)], rwin,
                                   sem_ld)
            ld2 = pltpu.async_copy(cols_hbm.at[pl.ds(off, WBUF)], cwin,
                                   sem_ld)
            ld3 = pltpu.async_copy(vals_hbm.at[pl.ds(off, WBUF)], vwin,
                                   sem_ld)
            ld1.wait()
            ld2.wait()
            ld3.wait()
            n_j = (jnp.maximum(hi - lo, 0) + 15) // 16

            @pl.loop(0, n_j)
            def _scat(j):
                t = j * 16
                rv = rwin[pl.ds(delta + t, 16)]
                cv = cwin[pl.ds(delta + t, 16)]
                vv = vwin[pl.ds(delta + t, 16)]
                g = lo + t + lax.broadcasted_iota(jnp.int32, (16,), 0)
                lrow = rv - rbase
                mask = ((g < hi) & (lrow >= 0) & (lrow < CROWS))
                plsc.store_scatter(dense, [lrow, cv], vv, mask=mask)

            pltpu.async_copy(dense, w_hbm.at[pl.ds(r0, CROWS)],
                             sem_o).wait()

            @pl.loop(0, n_j)
            def _cln(j):
                t = j * 16
                rv = rwin[pl.ds(delta + t, 16)]
                cv = cwin[pl.ds(delta + t, 16)]
                g = lo + t + lax.broadcasted_iota(jnp.int32, (16,), 0)
                lrow = rv - rbase
                mask = ((g < hi) & (lrow >= 0) & (lrow < CROWS))
                plsc.store_scatter(dense, [lrow, cv],
                                   jnp.zeros((16,), jnp.float32), mask=mask)

        @pl.when(nwin > 1)
        def _slow():
            with jax.named_scope("sc_scatter"):
                _windows(c, lo, hi, lambda v: v)
            with jax.named_scope("sc_dma_out"):
                pltpu.async_copy(dense, w_hbm.at[pl.ds(r0, CROWS)],
                                 sem_o).wait()
            with jax.named_scope("sc_clean"):
                _windows(c, lo, hi,
                         lambda v: jnp.zeros((16,), jnp.float32))


def _densify(rows_p, cols_p, vals_p, coarse, nnz):
    mesh = plsc.VectorSubcoreMesh(core_axis_name="c", subcore_axis_name="s")
    return pl.kernel(
        functools.partial(_scatter_body, nnz),
        out_type=jax.ShapeDtypeStruct((OUT_F, IN_F), jnp.float32),
        mesh=mesh,
        compiler_params=pltpu.CompilerParams(needs_layout_passes=False),
        scratch_types=[
            pltpu.VMEM((SBUF,), jnp.int32),
            pltpu.VMEM((CROWS, IN_F), jnp.float32),
            pltpu.VMEM((WBUF,), jnp.int32),
            pltpu.VMEM((WBUF,), jnp.int32),
            pltpu.VMEM((WBUF,), jnp.float32),
            pltpu.SemaphoreType.DMA,
            pltpu.SemaphoreType.DMA,
        ],
    )(rows_p, cols_p, vals_p, coarse)


def _mm_body(x_ref, w_ref, b_ref, o_ref):
    acc = lax.dot_general(
        x_ref[...], w_ref[...],
        (((1,), (1,)), ((), ())),
        preferred_element_type=jnp.float32)
    o_ref[...] = acc + b_ref[...][None, :]


def _matmul(x, w, bias, batch):
    nb = 512
    return pl.pallas_call(
        _mm_body,
        grid=(OUT_F // nb,),
        in_specs=[
            pl.BlockSpec((batch, IN_F), lambda j: (0, 0)),
            pl.BlockSpec((nb, IN_F), lambda j: (j, 0)),
            pl.BlockSpec((nb,), lambda j: (j,)),
        ],
        out_specs=pl.BlockSpec((batch, nb), lambda j: (0, j)),
        out_shape=jax.ShapeDtypeStruct((batch, OUT_F), jnp.float32),
    )(x, w, bias)


def kernel(x, sparse_indices, sparse_values, bias):
    orig_shape = x.shape
    x2d = x.reshape(-1, IN_F)
    batch = x2d.shape[0]

    nnz = sparse_values.shape[0]
    padn = -(-(nnz + 3 * WIN) // 131072) * 131072
    rows_p, cols_p, vals_p = _split(sparse_indices, sparse_values, padn)
    ns = -(-nnz // SAMP)
    pos2d = (jnp.arange(ns, dtype=jnp.int32)[:, None] * SAMP
             + jnp.arange(SAMP, dtype=jnp.int32)[None, :])
    r2d = rows_p[:ns * SAMP].reshape(ns, SAMP)
    sample = jnp.min(jnp.where(pos2d < nnz, r2d, OUT_F), axis=1)
    rbounds = jnp.arange(NCH + 1, dtype=jnp.int32) * CROWS
    coarse = jnp.sum(sample[None, :] < rbounds[:, None],
                     axis=1).astype(jnp.int32)
    coarse_p = jnp.concatenate(
        [coarse, jnp.zeros((SBUF - NCH - 1,), jnp.int32)])

    w = _densify(rows_p, cols_p, vals_p, coarse_p, nnz)
    y = _matmul(x2d, w, bias, batch)
    return y.reshape(*orig_shape[:-1], OUT_F).astype(x.dtype)


# speedup vs baseline: 1.3847x; 1.0000x over previous
import functools

import jax
import jax.numpy as jnp
from jax import lax
from jax.experimental import pallas as pl
from jax.experimental.pallas import tpu as pltpu
from jax.experimental.pallas import tpu_sc as plsc

IN_F = 4096
OUT_F = 4096
TOTAL = IN_F * OUT_F

NW = 32
CSZ = 65536
CROWS = CSZ // IN_F
SAMP = 512
NCH = TOTAL // CSZ
NSUB = NCH // NW
WIN = 8192
WBUF = WIN + 16
SBUF = ((NCH + 1 + 31) // 16) * 16


def _sel(buf, i):
    return buf[pl.ds(i, 16)][0]


def _split_body(idx_ref, v_ref, r_ref, c_ref, vo_ref):
    r_ref[...] = idx_ref[0, :]
    c_ref[...] = idx_ref[1, :]
    vo_ref[...] = v_ref[...]


def _split(sparse_indices, sparse_values, padn):
    bn = 131072
    grid = (padn // bn,)
    return pl.pallas_call(
        _split_body,
        grid=grid,
        in_specs=[
            pl.BlockSpec((2, bn), lambda j: (0, j)),
            pl.BlockSpec((bn,), lambda j: (j,)),
        ],
        out_specs=[
            pl.BlockSpec((bn,), lambda j: (j,)),
            pl.BlockSpec((bn,), lambda j: (j,)),
            pl.BlockSpec((bn,), lambda j: (j,)),
        ],
        out_shape=[
            jax.ShapeDtypeStruct((padn,), sparse_indices.dtype),
            jax.ShapeDtypeStruct((padn,), sparse_indices.dtype),
            jax.ShapeDtypeStruct((padn,), sparse_values.dtype),
        ],
    )(sparse_indices, sparse_values)


def _scatter_body(nnz, rows_hbm, cols_hbm, vals_hbm, coarse_hbm, w_hbm,
                  sbuf, dense, rwin, cwin, vwin, sem_ld, sem_o):
    w = lax.axis_index("s") * 2 + lax.axis_index("c")

    pltpu.sync_copy(coarse_hbm, sbuf)

    @pl.loop(0, CROWS)
    def _zero_row(r):
        @pl.loop(0, IN_F // 16)
        def _zero_init(i):
            dense[r, pl.ds(i * 16, 16)] = jnp.zeros((16,), jnp.float32)

    def _windows(c, start, end, value_of):
        astart = jnp.bitwise_and(start, jnp.int32(-8))
        delta = start - astart
        rbase = c * CROWS
        nwin = (end - start + WIN - 1) // WIN

        def _win(m, carry):
            off = pl.multiple_of(astart + m * WIN, 8)
            ld1 = pltpu.async_copy(rows_hbm.at[pl.ds(off, WBUF)], rwin,
                                   sem_ld)
            ld2 = pltpu.async_copy(cols_hbm.at[pl.ds(off, WBUF)], cwin,
                                   sem_ld)
            ld3 = pltpu.async_copy(vals_hbm.at[pl.ds(off, WBUF)], vwin,
                                   sem_ld)
            ld1.wait()
            ld2.wait()
            ld3.wait()
            rem = end - start - m * WIN
            n_j = (jnp.minimum(rem, WIN) + 15) // 16
            wbase = start + m * WIN

            @pl.loop(0, n_j)
            def _scat(j):
                t = j * 16
                rv = rwin[pl.ds(delta + t, 16)]
                cv = cwin[pl.ds(delta + t, 16)]
                vv = vwin[pl.ds(delta + t, 16)]
                g = wbase + t + lax.broadcasted_iota(jnp.int32, (16,), 0)
                lrow = rv - rbase
                mask = ((g < end) & (lrow >= 0) & (lrow < CROWS))
                plsc.store_scatter(dense, [lrow, cv], value_of(vv),
                                   mask=mask)

            return carry

        lax.fori_loop(0, nwin, _win, 0)

    @pl.loop(0, NSUB)
    def _sub(s):
        c = w * NSUB + s
        co0 = _sel(sbuf, c)
        co1 = _sel(sbuf, c + 1)
        lo = jnp.maximum(co0 - 1, 0) * SAMP
        hi = jnp.minimum(co1 * SAMP, nnz)
        r0 = pl.multiple_of(c * CROWS, CROWS)
        nwin = (hi - lo + WIN - 1) // WIN

        @pl.when(nwin <= 1)
        def _fast():
            astart = jnp.bitwise_and(lo, jnp.int32(-8))
            delta = lo - astart
            rbase = c * CROWS
            off = pl.multiple_of(astart, 8)
            ld1 = pltpu.async_copy(rows_hbm.at[pl.ds(off, WBUF)], rwin,
                                   sem_ld)
            ld2 = pltpu.async_copy(cols_hbm.at[pl.ds(off, WBUF)], cwin,
                                   sem_ld)
            ld3 = pltpu.async_copy(vals_hbm.at[pl.ds(off, WBUF)], vwin,
                                   sem_ld)
            ld1.wait()
            ld2.wait()
            ld3.wait()
            n_j = (jnp.maximum(hi - lo, 0) + 15) // 16

            @pl.loop(0, n_j)
            def _scat(j):
                t = j * 16
                rv = rwin[pl.ds(delta + t, 16)]
                cv = cwin[pl.ds(delta + t, 16)]
                vv = vwin[pl.ds(delta + t, 16)]
                g = lo + t + lax.broadcasted_iota(jnp.int32, (16,), 0)
                lrow = rv - rbase
                mask = ((g < hi) & (lrow >= 0) & (lrow < CROWS))
                plsc.store_scatter(dense, [lrow, cv], vv, mask=mask)

            pltpu.async_copy(dense, w_hbm.at[pl.ds(r0, CROWS)],
                             sem_o).wait()

            @pl.loop(0, n_j)
            def _cln(j):
                t = j * 16
                rv = rwin[pl.ds(delta + t, 16)]
                cv = cwin[pl.ds(delta + t, 16)]
                g = lo + t + lax.broadcasted_iota(jnp.int32, (16,), 0)
                lrow = rv - rbase
                mask = ((g < hi) & (lrow >= 0) & (lrow < CROWS))
                plsc.store_scatter(dense, [lrow, cv],
                                   jnp.zeros((16,), jnp.float32), mask=mask)

        @pl.when(nwin > 1)
        def _slow():
            with jax.named_scope("sc_scatter"):
                _windows(c, lo, hi, lambda v: v)
            with jax.named_scope("sc_dma_out"):
                pltpu.async_copy(dense, w_hbm.at[pl.ds(r0, CROWS)],
                                 sem_o).wait()
            with jax.named_scope("sc_clean"):
                _windows(c, lo, hi,
                         lambda v: jnp.zeros((16,), jnp.float32))


def _densify(rows_p, cols_p, vals_p, coarse, nnz):
    mesh = plsc.VectorSubcoreMesh(core_axis_name="c", subcore_axis_name="s")
    return pl.kernel(
        functools.partial(_scatter_body, nnz),
        out_type=jax.ShapeDtypeStruct((OUT_F, IN_F), jnp.float32),
        mesh=mesh,
        compiler_params=pltpu.CompilerParams(needs_layout_passes=False),
        scratch_types=[
            pltpu.VMEM((SBUF,), jnp.int32),
            pltpu.VMEM((CROWS, IN_F), jnp.float32),
            pltpu.VMEM((WBUF,), jnp.int32),
            pltpu.VMEM((WBUF,), jnp.int32),
            pltpu.VMEM((WBUF,), jnp.float32),
            pltpu.SemaphoreType.DMA,
            pltpu.SemaphoreType.DMA,
        ],
    )(rows_p, cols_p, vals_p, coarse)


def _mm_body(x_ref, w_ref, b_ref, o_ref):
    acc = lax.dot_general(
        x_ref[...], w_ref[...],
        (((1,), (1,)), ((), ())),
        preferred_element_type=jnp.float32)
    o_ref[...] = acc + b_ref[...][None, :]


def _matmul(x, w, bias, batch):
    nb = 512
    return pl.pallas_call(
        _mm_body,
        grid=(OUT_F // nb,),
        in_specs=[
            pl.BlockSpec((batch, IN_F), lambda j: (0, 0)),
            pl.BlockSpec((nb, IN_F), lambda j: (j, 0)),
            pl.BlockSpec((nb,), lambda j: (j,)),
        ],
        out_specs=pl.BlockSpec((batch, nb), lambda j: (0, j)),
        out_shape=jax.ShapeDtypeStruct((batch, OUT_F), jnp.float32),
    )(x, w, bias)


def kernel(x, sparse_indices, sparse_values, bias):
    orig_shape = x.shape
    x2d = x.reshape(-1, IN_F)
    batch = x2d.shape[0]

    nnz = sparse_values.shape[0]
    padn = -(-(nnz + 3 * WIN) // 131072) * 131072
    rows_p, cols_p, vals_p = _split(sparse_indices, sparse_values, padn)
    ns = -(-nnz // SAMP)
    pos2d = (jnp.arange(ns, dtype=jnp.int32)[:, None] * SAMP
             + jnp.arange(SAMP, dtype=jnp.int32)[None, :])
    r2d = rows_p[:ns * SAMP].reshape(ns, SAMP)
    sample = jnp.min(jnp.where(pos2d < nnz, r2d, OUT_F), axis=1)
    rbounds = jnp.arange(NCH + 1, dtype=jnp.int32) * CROWS
    coarse = jnp.sum(sample[None, :] < rbounds[:, None],
                     axis=1).astype(jnp.int32)
    coarse_p = jnp.concatenate(
        [coarse, jnp.zeros((SBUF - NCH - 1,), jnp.int32)])

    w = _densify(rows_p, cols_p, vals_p, coarse_p, nnz)
    y = _matmul(x2d, w, bias, batch)
    return y.reshape(*orig_shape[:-1], OUT_F).astype(x.dtype)
